# initial kernel scaffold (unmeasured)
import jax
import jax.numpy as jnp
from jax import lax
from jax.experimental import pallas as pl
from jax.experimental.pallas import tpu as pltpu

N_DEV = 8
B, Sq, Hq, Dh = 4, 256, 8, 128
D = Hq * Dh
SCALE = 0.08838834764831843


def kernel(x, Wq, Wo, K_ext, V_ext):
    x2 = x.reshape(B * Sq, D)

    def body(x_ref, wq_ref, wo_ref, k_ref, v_ref, out_ref,
             q_ref, oacc_ref, lacc_ref, commo_ref, comml_ref,
             sendo, recvo, sendl, recvl):
        my = lax.axis_index("i")
        left = lax.rem(my + N_DEV - 1, N_DEV)
        right = lax.rem(my + 1, N_DEV)

        barrier_sem = pltpu.get_barrier_semaphore()
        for nbr in (left, right):
            pl.semaphore_signal(
                barrier_sem, inc=1,
                device_id=(nbr,), device_id_type=pl.DeviceIdType.MESH,
            )
        pl.semaphore_wait(barrier_sem, 2)

        q_ref[...] = jnp.dot(
            x_ref[...], wq_ref[...], preferred_element_type=jnp.float32
        )

        for b in range(B):
            for h in range(Hq):
                qbh = q_ref[b * Sq:(b + 1) * Sq, h * Dh:(h + 1) * Dh]
                kbh = k_ref[b, :, h, :]
                vbh = v_ref[b, :, h, :]
                s = lax.dot_general(
                    qbh, kbh, (((1,), (1,)), ((), ())),
                    preferred_element_type=jnp.float32,
                ) * SCALE
                p = jnp.exp(s)
                col = b * Hq + h
                lacc_ref[:, col:col + 1] = jnp.sum(p, axis=1, keepdims=True)
                oacc_ref[b * Sq:(b + 1) * Sq, h * Dh:(h + 1) * Dh] = (
                    lax.dot_general(
                        p, vbh, (((1,), (0,)), ((), ())),
                        preferred_element_type=jnp.float32,
                    )
                )

        commo_ref[0] = oacc_ref[...]
        comml_ref[0] = lacc_ref[...]
        for h in range(N_DEV - 1):
            ss, rs = h % 2, (h + 1) % 2
            rdma_o = pltpu.make_async_remote_copy(
                src_ref=commo_ref.at[ss], dst_ref=commo_ref.at[rs],
                send_sem=sendo.at[ss], recv_sem=recvo.at[rs],
                device_id=(right,), device_id_type=pl.DeviceIdType.MESH,
            )
            rdma_l = pltpu.make_async_remote_copy(
                src_ref=comml_ref.at[ss], dst_ref=comml_ref.at[rs],
                send_sem=sendl.at[ss], recv_sem=recvl.at[rs],
                device_id=(right,), device_id_type=pl.DeviceIdType.MESH,
            )
            rdma_o.start()
            rdma_l.start()
            rdma_o.wait()
            rdma_l.wait()
            oacc_ref[...] = oacc_ref[...] + commo_ref[rs]
            lacc_ref[...] = lacc_ref[...] + comml_ref[rs]

        for b in range(B):
            for h in range(Hq):
                col = b * Hq + h
                oacc_ref[b * Sq:(b + 1) * Sq, h * Dh:(h + 1) * Dh] = (
                    oacc_ref[b * Sq:(b + 1) * Sq, h * Dh:(h + 1) * Dh]
                    / lacc_ref[:, col:col + 1]
                )
        out_ref[...] = jnp.dot(
            oacc_ref[...], wo_ref[...], preferred_element_type=jnp.float32
        )

    out = pl.pallas_call(
        body,
        out_shape=jax.ShapeDtypeStruct((B * Sq, D), jnp.float32),
        in_specs=[pl.BlockSpec(memory_space=pltpu.VMEM)] * 5,
        out_specs=pl.BlockSpec(memory_space=pltpu.VMEM),
        scratch_shapes=[
            pltpu.VMEM((B * Sq, D), jnp.float32),
            pltpu.VMEM((B * Sq, D), jnp.float32),
            pltpu.VMEM((Sq, B * Hq), jnp.float32),
            pltpu.VMEM((2, B * Sq, D), jnp.float32),
            pltpu.VMEM((2, Sq, B * Hq), jnp.float32),
            pltpu.SemaphoreType.DMA((2,)),
            pltpu.SemaphoreType.DMA((2,)),
            pltpu.SemaphoreType.DMA((2,)),
            pltpu.SemaphoreType.DMA((2,)),
        ],
        compiler_params=pltpu.CompilerParams(collective_id=0),
    )(x2, Wq, Wo, K_ext, V_ext)
    return out.reshape(B, Sq, D)


# baseline (device time: 397279 ns/iter reference)
import jax
import jax.numpy as jnp
from jax import lax
from jax.experimental import pallas as pl
from jax.experimental.pallas import tpu as pltpu

N_DEV = 8
B, Sq, Hq, Dh = 4, 256, 8, 128
D = Hq * Dh
SCALE = 0.08838834764831843


def kernel(x, Wq, Wo, K_ext, V_ext):
    x2 = x.reshape(B * Sq, D)

    def body(x_ref, wq_ref, wo_ref, k_ref, v_ref, out_ref,
             q_ref, oacc_ref, lacc_ref, commo_ref, comml_ref,
             sendo, recvo, sendl, recvl):
        my = lax.axis_index("i")
        left = lax.rem(my + N_DEV - 1, N_DEV)
        right = lax.rem(my + 1, N_DEV)

        barrier_sem = pltpu.get_barrier_semaphore()
        for nbr in (left, right):
            pl.semaphore_signal(
                barrier_sem, inc=1,
                device_id=(nbr,), device_id_type=pl.DeviceIdType.MESH,
            )
        pl.semaphore_wait(barrier_sem, 2)

        q_ref[...] = jnp.dot(
            x_ref[...], wq_ref[...], preferred_element_type=jnp.float32
        )

        for b in range(B):
            for h in range(Hq):
                qbh = q_ref[b * Sq:(b + 1) * Sq, h * Dh:(h + 1) * Dh]
                kbh = k_ref[b, :, h, :]
                vbh = v_ref[b, :, h, :]
                s = lax.dot_general(
                    qbh, kbh, (((1,), (1,)), ((), ())),
                    preferred_element_type=jnp.float32,
                ) * SCALE
                p = jnp.exp(s)
                col = b * Hq + h
                lacc_ref[:, col:col + 1] = jnp.sum(p, axis=1, keepdims=True)
                oacc_ref[b * Sq:(b + 1) * Sq, h * Dh:(h + 1) * Dh] = (
                    lax.dot_general(
                        p, vbh, (((1,), (0,)), ((), ())),
                        preferred_element_type=jnp.float32,
                    )
                )

        commo_ref[0] = oacc_ref[...]
        comml_ref[0] = lacc_ref[...]
        for h in range(N_DEV - 1):
            ss, rs = h % 2, (h + 1) % 2
            rdma_o = pltpu.make_async_remote_copy(
                src_ref=commo_ref.at[ss], dst_ref=commo_ref.at[rs],
                send_sem=sendo.at[ss], recv_sem=recvo.at[rs],
                device_id=(right,), device_id_type=pl.DeviceIdType.MESH,
            )
            rdma_l = pltpu.make_async_remote_copy(
                src_ref=comml_ref.at[ss], dst_ref=comml_ref.at[rs],
                send_sem=sendl.at[ss], recv_sem=recvl.at[rs],
                device_id=(right,), device_id_type=pl.DeviceIdType.MESH,
            )
            rdma_o.start()
            rdma_l.start()
            rdma_o.wait()
            rdma_l.wait()
            oacc_ref[...] = oacc_ref[...] + commo_ref[rs]
            lacc_ref[...] = lacc_ref[...] + comml_ref[rs]

        for b in range(B):
            for h in range(Hq):
                col = b * Hq + h
                oacc_ref[b * Sq:(b + 1) * Sq, h * Dh:(h + 1) * Dh] = (
                    oacc_ref[b * Sq:(b + 1) * Sq, h * Dh:(h + 1) * Dh]
                    / lacc_ref[:, col:col + 1]
                )
        out_ref[...] = jnp.dot(
            oacc_ref[...], wo_ref[...], preferred_element_type=jnp.float32
        )

    out = pl.pallas_call(
        body,
        out_shape=jax.ShapeDtypeStruct((B * Sq, D), jnp.float32),
        in_specs=[pl.BlockSpec(memory_space=pltpu.VMEM)] * 5,
        out_specs=pl.BlockSpec(memory_space=pltpu.VMEM),
        scratch_shapes=[
            pltpu.VMEM((B * Sq, D), jnp.float32),
            pltpu.VMEM((B * Sq, D), jnp.float32),
            pltpu.VMEM((Sq, B * Hq), jnp.float32),
            pltpu.VMEM((2, B * Sq, D), jnp.float32),
            pltpu.VMEM((2, Sq, B * Hq), jnp.float32),
            pltpu.SemaphoreType.DMA((2,)),
            pltpu.SemaphoreType.DMA((2,)),
            pltpu.SemaphoreType.DMA((2,)),
            pltpu.SemaphoreType.DMA((2,)),
        ],
        compiler_params=pltpu.CompilerParams(
            collective_id=0, vmem_limit_bytes=120 * 1024 * 1024
        ),
    )(x2, Wq, Wo, K_ext, V_ext)
    return out.reshape(B, Sq, D)


# device time: 171441 ns/iter; 2.3173x vs baseline; 2.3173x over previous
import jax
import jax.numpy as jnp
from jax import lax
from jax.experimental import pallas as pl
from jax.experimental.pallas import tpu as pltpu

N_DEV = 8
B, Sq, Hq, Dh = 4, 256, 8, 128
D = Hq * Dh
CH = (B * Sq) // N_DEV
SCALE = 0.08838834764831843


def kernel(x, Wq, Wo, K_ext, V_ext):
    x2 = x.reshape(B * Sq, D)

    def body(x_ref, wq_ref, wo_ref, k_ref, v_ref, out_ref,
             q_ref, oacc_ref, lacc_ref, commo_ref, comml_ref,
             sendo, recvo, sendl, recvl, sendo2, recvo2):
        my = lax.axis_index("i")
        left = lax.rem(my + N_DEV - 1, N_DEV)
        right = lax.rem(my + 1, N_DEV)

        barrier_sem = pltpu.get_barrier_semaphore()
        for nbr in (left, right):
            pl.semaphore_signal(
                barrier_sem, inc=1,
                device_id=(nbr,), device_id_type=pl.DeviceIdType.MESH,
            )
        pl.semaphore_wait(barrier_sem, 2)

        q_ref[...] = jnp.dot(
            x_ref[...], wq_ref[...], preferred_element_type=jnp.float32
        )

        for b in range(B):
            for h in range(Hq):
                qbh = q_ref[b * Sq:(b + 1) * Sq, h * Dh:(h + 1) * Dh]
                kbh = k_ref[b, :, h, :]
                vbh = v_ref[b, :, h, :]
                s = lax.dot_general(
                    qbh, kbh, (((1,), (1,)), ((), ())),
                    preferred_element_type=jnp.float32,
                ) * SCALE
                p = jnp.exp(s)
                col = b * Hq + h
                lacc_ref[:, col:col + 1] = jnp.sum(p, axis=1, keepdims=True)
                oacc_ref[b * Sq:(b + 1) * Sq, h * Dh:(h + 1) * Dh] = (
                    lax.dot_general(
                        p, vbh, (((1,), (0,)), ((), ())),
                        preferred_element_type=jnp.float32,
                    )
                )

        comml_ref[0] = lacc_ref[...]
        for t in range(N_DEV - 1):
            ss, rs = t % 2, (t + 1) % 2
            sc = lax.rem(my + N_DEV - t, N_DEV)
            rc = lax.rem(my + N_DEV - t - 1, N_DEV)
            rdma_o = pltpu.make_async_remote_copy(
                src_ref=oacc_ref.at[pl.ds(sc * CH, CH), :],
                dst_ref=commo_ref.at[rs],
                send_sem=sendo.at[t], recv_sem=recvo.at[t],
                device_id=(right,), device_id_type=pl.DeviceIdType.MESH,
            )
            rdma_l = pltpu.make_async_remote_copy(
                src_ref=comml_ref.at[ss], dst_ref=comml_ref.at[rs],
                send_sem=sendl.at[t], recv_sem=recvl.at[t],
                device_id=(right,), device_id_type=pl.DeviceIdType.MESH,
            )
            rdma_o.start()
            rdma_l.start()
            rdma_o.wait()
            rdma_l.wait()
            oacc_ref[pl.ds(rc * CH, CH), :] = (
                oacc_ref[pl.ds(rc * CH, CH), :] + commo_ref[rs]
            )
            comml_ref[rs] = comml_ref[rs] + lacc_ref[...]

        lacc_ref[...] = comml_ref[(N_DEV - 1) % 2]

        for t in range(N_DEV - 1):
            sc = lax.rem(my + 1 + N_DEV - t, N_DEV)
            rdma_o = pltpu.make_async_remote_copy(
                src_ref=oacc_ref.at[pl.ds(sc * CH, CH), :],
                dst_ref=oacc_ref.at[pl.ds(sc * CH, CH), :],
                send_sem=sendo2.at[t], recv_sem=recvo2.at[t],
                device_id=(right,), device_id_type=pl.DeviceIdType.MESH,
            )
            rdma_o.start()
            rdma_o.wait()

        for b in range(B):
            for h in range(Hq):
                col = b * Hq + h
                oacc_ref[b * Sq:(b + 1) * Sq, h * Dh:(h + 1) * Dh] = (
                    oacc_ref[b * Sq:(b + 1) * Sq, h * Dh:(h + 1) * Dh]
                    / lacc_ref[:, col:col + 1]
                )
        out_ref[...] = jnp.dot(
            oacc_ref[...], wo_ref[...], preferred_element_type=jnp.float32
        )

    out = pl.pallas_call(
        body,
        out_shape=jax.ShapeDtypeStruct((B * Sq, D), jnp.float32),
        in_specs=[pl.BlockSpec(memory_space=pltpu.VMEM)] * 5,
        out_specs=pl.BlockSpec(memory_space=pltpu.VMEM),
        scratch_shapes=[
            pltpu.VMEM((B * Sq, D), jnp.float32),
            pltpu.VMEM((B * Sq, D), jnp.float32),
            pltpu.VMEM((Sq, B * Hq), jnp.float32),
            pltpu.VMEM((2, CH, D), jnp.float32),
            pltpu.VMEM((2, Sq, B * Hq), jnp.float32),
            pltpu.SemaphoreType.DMA((N_DEV - 1,)),
            pltpu.SemaphoreType.DMA((N_DEV - 1,)),
            pltpu.SemaphoreType.DMA((N_DEV - 1,)),
            pltpu.SemaphoreType.DMA((N_DEV - 1,)),
            pltpu.SemaphoreType.DMA((N_DEV - 1,)),
            pltpu.SemaphoreType.DMA((N_DEV - 1,)),
        ],
        compiler_params=pltpu.CompilerParams(
            collective_id=0, vmem_limit_bytes=120 * 1024 * 1024
        ),
    )(x2, Wq, Wo, K_ext, V_ext)
    return out.reshape(B, Sq, D)


# device time: 68984 ns/iter; 5.7590x vs baseline; 2.4852x over previous
import jax
import jax.numpy as jnp
from jax import lax
from jax.experimental import pallas as pl
from jax.experimental.pallas import tpu as pltpu

N_DEV = 8
B, Sq, Hq, Dh = 4, 256, 8, 128
D = Hq * Dh
CH = (B * Sq) // N_DEV
SCALE = 0.08838834764831843


def _unrank(v):
    return jnp.where(v < 4, v, 11 - v)


def kernel(x, Wq, Wo, K_ext, V_ext):
    x2 = x.reshape(B * Sq, D)

    def body(x_ref, wq_ref, wo_ref, k_ref, v_ref, out_ref,
             oacc_ref, lacc_ref, commo_ref, comml_ref,
             sendo, recvo, sendl, recvl, sendo2, recvo2):
        my = lax.axis_index("i")
        r = _unrank(my)
        right = _unrank(lax.rem(r + 1, N_DEV))
        left = _unrank(lax.rem(r + N_DEV - 1, N_DEV))

        barrier_sem = pltpu.get_barrier_semaphore()
        for nbr in (left, right):
            pl.semaphore_signal(
                barrier_sem, inc=1,
                device_id=(nbr,), device_id_type=pl.DeviceIdType.MESH,
            )
        pl.semaphore_wait(barrier_sem, 2)

        def compute_chunk(c):
            r0 = c * CH
            bb = lax.div(c, 2)
            qc = jnp.dot(
                x_ref[pl.ds(r0, CH), :], wq_ref[...],
                preferred_element_type=jnp.float32,
            )
            for h in range(Hq):
                kbh = k_ref[bb, :, h, :]
                vbh = v_ref[bb, :, h, :]
                s = lax.dot_general(
                    qc[:, h * Dh:(h + 1) * Dh], kbh,
                    (((1,), (1,)), ((), ())),
                    preferred_element_type=jnp.float32,
                ) * SCALE
                p = jnp.exp(s)
                lacc_ref[pl.ds(r0, CH), h:h + 1] = jnp.sum(
                    p, axis=1, keepdims=True
                )
                oacc_ref[pl.ds(r0, CH), h * Dh:(h + 1) * Dh] = (
                    lax.dot_general(
                        p, vbh, (((1,), (0,)), ((), ())),
                        preferred_element_type=jnp.float32,
                    )
                )

        compute_chunk(r)
        for t in range(N_DEV - 1):
            slot = t % 2
            sc = lax.rem(r + N_DEV - t, N_DEV)
            rc = lax.rem(r + N_DEV - t - 1, N_DEV)
            rdma_o = pltpu.make_async_remote_copy(
                src_ref=oacc_ref.at[pl.ds(sc * CH, CH)],
                dst_ref=commo_ref.at[slot],
                send_sem=sendo.at[t], recv_sem=recvo.at[t],
                device_id=(right,), device_id_type=pl.DeviceIdType.MESH,
            )
            rdma_l = pltpu.make_async_remote_copy(
                src_ref=lacc_ref.at[pl.ds(sc * CH, CH)],
                dst_ref=comml_ref.at[slot],
                send_sem=sendl.at[t], recv_sem=recvl.at[t],
                device_id=(right,), device_id_type=pl.DeviceIdType.MESH,
            )
            rdma_o.start()
            rdma_l.start()
            compute_chunk(rc)
            rdma_o.wait()
            rdma_l.wait()
            oacc_ref[pl.ds(rc * CH, CH), :] = (
                oacc_ref[pl.ds(rc * CH, CH), :] + commo_ref[slot]
            )
            lacc_ref[pl.ds(rc * CH, CH), :] = (
                lacc_ref[pl.ds(rc * CH, CH), :] + comml_ref[slot]
            )

        own = lax.rem(r + 1, N_DEV)
        o0 = own * CH
        for h in range(Hq):
            oacc_ref[pl.ds(o0, CH), h * Dh:(h + 1) * Dh] = (
                oacc_ref[pl.ds(o0, CH), h * Dh:(h + 1) * Dh]
                / lacc_ref[pl.ds(o0, CH), h:h + 1]
            )

        for t in range(N_DEV - 1):
            sc = lax.rem(r + 1 + N_DEV - t, N_DEV)
            rdma_o = pltpu.make_async_remote_copy(
                src_ref=oacc_ref.at[pl.ds(sc * CH, CH)],
                dst_ref=oacc_ref.at[pl.ds(sc * CH, CH)],
                send_sem=sendo2.at[t], recv_sem=recvo2.at[t],
                device_id=(right,), device_id_type=pl.DeviceIdType.MESH,
            )
            rdma_o.start()
            out_ref[pl.ds(sc * CH, CH), :] = jnp.dot(
                oacc_ref[pl.ds(sc * CH, CH), :], wo_ref[...],
                preferred_element_type=jnp.float32,
            )
            rdma_o.wait()
        last = lax.rem(r + 2, N_DEV)
        out_ref[pl.ds(last * CH, CH), :] = jnp.dot(
            oacc_ref[pl.ds(last * CH, CH), :], wo_ref[...],
            preferred_element_type=jnp.float32,
        )

    out = pl.pallas_call(
        body,
        out_shape=jax.ShapeDtypeStruct((B * Sq, D), jnp.float32),
        in_specs=[pl.BlockSpec(memory_space=pltpu.VMEM)] * 5,
        out_specs=pl.BlockSpec(memory_space=pltpu.VMEM),
        scratch_shapes=[
            pltpu.VMEM((B * Sq, D), jnp.float32),
            pltpu.VMEM((B * Sq, Hq), jnp.float32),
            pltpu.VMEM((2, CH, D), jnp.float32),
            pltpu.VMEM((2, CH, Hq), jnp.float32),
            pltpu.SemaphoreType.DMA((N_DEV - 1,)),
            pltpu.SemaphoreType.DMA((N_DEV - 1,)),
            pltpu.SemaphoreType.DMA((N_DEV - 1,)),
            pltpu.SemaphoreType.DMA((N_DEV - 1,)),
            pltpu.SemaphoreType.DMA((N_DEV - 1,)),
            pltpu.SemaphoreType.DMA((N_DEV - 1,)),
        ],
        compiler_params=pltpu.CompilerParams(
            collective_id=0, vmem_limit_bytes=120 * 1024 * 1024
        ),
    )(x2, Wq, Wo, K_ext, V_ext)
    return out.reshape(B, Sq, D)
